# baseline (speedup 1.0000x reference)
import functools

import jax
import jax.numpy as jnp
from jax import lax
from jax.experimental import pallas as pl
from jax.experimental.pallas import tpu as pltpu
from jax.experimental.pallas import tpu_sc as plsc

B, S, V, D = 4096, 50, 1000000, 32
N = B * S

_info = plsc.get_sparse_core_info()
NC, NS = _info.num_cores, _info.num_subcores
NW = NC * NS
PER_W = N // NW
CH = 1600
NCH = PER_W // CH


def _sc_gather():
  mesh = plsc.VectorSubcoreMesh(core_axis_name="c", subcore_axis_name="s")

  @functools.partial(
      pl.kernel,
      mesh=mesh,
      compiler_params=pltpu.CompilerParams(use_tc_tiling_on_sc=False),
      out_type=jax.ShapeDtypeStruct((4 * N, D), jnp.float32),
      scratch_types=[
          pltpu.VMEM((CH,), jnp.int32),
          pltpu.VMEM((CH,), jnp.int32),
          pltpu.VMEM((CH, D), jnp.float32),
          pltpu.VMEM((CH, D), jnp.float32),
          pltpu.SemaphoreType.DMA,
          pltpu.SemaphoreType.DMA,
          pltpu.SemaphoreType.DMA,
      ],
  )
  def k(w_src, w_tgt, idx_src, idx_hyp, idx_r0, idx_r1, out,
        idx_v0, idx_v1, rows_v0, rows_v1, gsem, wsem0, wsem1):
    wid = lax.axis_index("s") * NC + lax.axis_index("c")
    idx_v = (idx_v0, idx_v1)
    rows_v = (rows_v0, rows_v1)
    wsem = (wsem0, wsem1)

    segs = ((w_src, idx_src), (w_tgt, idx_hyp), (w_tgt, idx_r0),
            (w_tgt, idx_r1))
    writes = [None, None]
    step = 0
    for seg, (table, idx_hbm) in enumerate(segs):
      for c in range(NCH):
        b = step % 2
        base = wid * PER_W + c * CH
        if writes[b] is not None:
          writes[b].wait()
        pltpu.sync_copy(idx_hbm.at[pl.ds(base, CH)], idx_v[b])
        pltpu.async_copy(table.at[idx_v[b]], rows_v[b], gsem).wait()
        writes[b] = pltpu.async_copy(
            rows_v[b], out.at[pl.ds(seg * N + base, CH)], wsem[b])
        step += 1
    for w in writes:
      if w is not None:
        w.wait()

  return k


_TBLK = 8192
_TQ = _TBLK // 4
_TGRID = (V + _TBLK - 1) // _TBLK
_VPAD = _TGRID * _TBLK


def _table_t_body(wt_ref, out_ref):
  eye = jnp.eye(D, dtype=jnp.float32)
  parts = [
      lax.dot_general(wt_ref[:, c * _TQ:(c + 1) * _TQ], eye,
                      (((0,), (0,)), ((), ())),
                      preferred_element_type=jnp.float32, precision=lax.Precision.HIGHEST)
      for c in range(4)
  ]
  out_ref[...] = jnp.concatenate(parts, axis=1)


def _transpose_table(wt):
  return pl.pallas_call(
      _table_t_body,
      grid=(_TGRID,),
      in_specs=[pl.BlockSpec((D, _TBLK), lambda j: (0, j))],
      out_specs=pl.BlockSpec((_TQ, 4 * D), lambda j: (j, 0)),
      out_shape=jax.ShapeDtypeStruct((_VPAD // 4, 4 * D), jnp.float32),
  )(wt)


def _out_t_body(g_ref, out_ref):
  g = g_ref[0, 0]
  eye = jnp.eye(D, dtype=jnp.float32)
  parts = [
      lax.dot_general(eye, g[:, j * D:(j + 1) * D],
                      (((1,), (1,)), ((), ())),
                      preferred_element_type=jnp.float32, precision=lax.Precision.HIGHEST)
      for j in range(4)
  ]
  out_ref[0, 0] = jnp.concatenate(parts, axis=1)


def _transpose_out(g4):
  return pl.pallas_call(
      _out_t_body,
      grid=(4, S),
      in_specs=[pl.BlockSpec((1, 1, B // 4, 4 * D), lambda i, s: (i, s, 0, 0))],
      out_specs=pl.BlockSpec((1, 1, D, B), lambda i, s: (i, s, 0, 0)),
      out_shape=jax.ShapeDtypeStruct((4, S, D, B), jnp.float32),
  )(g4)


def _prep_idx(ids):
  a = ids.T.astype(jnp.int32)
  u = a & (_TBLK - 1)
  a2 = (a - u) + ((u & (_TQ - 1)) << 2) + (u >> 11)
  return a2.reshape(S, 4, B // 4).transpose(0, 2, 1).reshape(N)


def _mask_body(a_ref, b_ref, c_ref, d_ref, oa_ref, ob_ref, oc_ref, od_ref):
  oa_ref[...] = a_ref[...] == 0
  ob_ref[...] = b_ref[...] == 0
  oc_ref[...] = c_ref[...] == 0
  od_ref[...] = d_ref[...] == 0


def kernel(sources_input_ids, sources_attention_mask,
           hypotheses_input_ids, hypotheses_attention_mask,
           ref0_input_ids, ref0_attention_mask,
           ref1_input_ids, ref1_attention_mask,
           W_src, W_tgt):
  idx_src = _prep_idx(sources_input_ids)
  idx_hyp = _prep_idx(hypotheses_input_ids)
  idx_r0 = _prep_idx(ref0_input_ids)
  idx_r1 = _prep_idx(ref1_input_ids)

  ws = _transpose_table(W_src.T).reshape(_VPAD, D)
  wt = _transpose_table(W_tgt.T).reshape(_VPAD, D)

  gathered = _sc_gather()(ws, wt, idx_src, idx_hyp, idx_r0, idx_r1)
  outt = _transpose_out(gathered.reshape(4, S, B // 4, 4 * D))

  embedded_sources = outt[0].transpose(2, 0, 1)
  embedded_hypotheses = outt[1].transpose(2, 0, 1)
  embedded_ref0 = outt[2].transpose(2, 0, 1)
  embedded_ref1 = outt[3].transpose(2, 0, 1)

  inv = pl.pallas_call(
      _mask_body,
      out_shape=[jax.ShapeDtypeStruct((S, B), jnp.bool_)] * 4,
  )(sources_attention_mask.T, hypotheses_attention_mask.T,
    ref0_attention_mask.T, ref1_attention_mask.T)

  return (embedded_sources, embedded_hypotheses, embedded_ref0, embedded_ref1,
          inv[0].T, inv[1].T, inv[2].T, inv[3].T)


# speedup vs baseline: 1.8165x; 1.8165x over previous
import functools

import jax
import jax.numpy as jnp
from jax import lax
from jax.experimental import pallas as pl
from jax.experimental.pallas import tpu as pltpu
from jax.experimental.pallas import tpu_sc as plsc

B, S, V, D = 4096, 50, 1000000, 32
N = B * S

_info = plsc.get_sparse_core_info()
NC, NS = _info.num_cores, _info.num_subcores
NW = NC * NS
PER_W = N // NW
CH = 1600
NCH = PER_W // CH


def _sc_gather():
  mesh = plsc.VectorSubcoreMesh(core_axis_name="c", subcore_axis_name="s")

  @functools.partial(
      pl.kernel,
      mesh=mesh,
      compiler_params=pltpu.CompilerParams(use_tc_tiling_on_sc=False),
      out_type=jax.ShapeDtypeStruct((4 * N, D), jnp.float32),
      scratch_types=[
          pltpu.VMEM((CH,), jnp.int32),
          pltpu.VMEM((CH,), jnp.int32),
          pltpu.VMEM((CH, D), jnp.float32),
          pltpu.VMEM((CH, D), jnp.float32),
          pltpu.SemaphoreType.DMA,
          pltpu.SemaphoreType.DMA,
          pltpu.SemaphoreType.DMA,
      ],
  )
  def k(w_src, w_tgt, idx_src, idx_hyp, idx_r0, idx_r1, out,
        idx_v0, idx_v1, rows_v0, rows_v1, gsem, wsem0, wsem1):
    wid = lax.axis_index("s") * NC + lax.axis_index("c")
    idx_v = (idx_v0, idx_v1)
    rows_v = (rows_v0, rows_v1)
    wsem = (wsem0, wsem1)

    segs = ((w_src, idx_src), (w_tgt, idx_hyp), (w_tgt, idx_r0),
            (w_tgt, idx_r1))
    writes = [None, None]
    step = 0
    for seg, (table, idx_hbm) in enumerate(segs):
      for c in range(NCH):
        b = step % 2
        base = wid * PER_W + c * CH
        if writes[b] is not None:
          writes[b].wait()
        pltpu.sync_copy(idx_hbm.at[pl.ds(base, CH)], idx_v[b])
        pltpu.async_copy(table.at[idx_v[b]], rows_v[b], gsem).wait()
        writes[b] = pltpu.async_copy(
            rows_v[b], out.at[pl.ds(seg * N + base, CH)], wsem[b])
        step += 1
    for w in writes:
      if w is not None:
        w.wait()

  return k


_TBLK = 8192
_TQ = _TBLK // 4
_TGRID = (V + _TBLK - 1) // _TBLK
_VPAD = _TGRID * _TBLK


def _table_t_body(wt_ref, out_ref):
  eye = jnp.eye(D, dtype=jnp.float32)
  parts = [
      lax.dot_general(wt_ref[:, c * _TQ:(c + 1) * _TQ], eye,
                      (((0,), (0,)), ((), ())),
                      preferred_element_type=jnp.float32)
      for c in range(4)
  ]
  out_ref[...] = jnp.concatenate(parts, axis=1)


def _transpose_table(wt):
  return pl.pallas_call(
      _table_t_body,
      grid=(_TGRID,),
      in_specs=[pl.BlockSpec((D, _TBLK), lambda j: (0, j))],
      out_specs=pl.BlockSpec((_TQ, 4 * D), lambda j: (j, 0)),
      out_shape=jax.ShapeDtypeStruct((_VPAD // 4, 4 * D), jnp.float32),
  )(wt)


def _out_t_body(g_ref, out_ref):
  g = g_ref[0, 0]
  eye = jnp.eye(D, dtype=jnp.float32)
  parts = [
      lax.dot_general(eye, g[:, j * D:(j + 1) * D],
                      (((1,), (1,)), ((), ())),
                      preferred_element_type=jnp.float32)
      for j in range(4)
  ]
  out_ref[0, 0] = jnp.concatenate(parts, axis=1)


def _transpose_out(g4):
  return pl.pallas_call(
      _out_t_body,
      grid=(4, S),
      in_specs=[pl.BlockSpec((1, 1, B // 4, 4 * D), lambda i, s: (i, s, 0, 0))],
      out_specs=pl.BlockSpec((1, 1, D, B), lambda i, s: (i, s, 0, 0)),
      out_shape=jax.ShapeDtypeStruct((4, S, D, B), jnp.float32),
  )(g4)


def _prep_idx(ids):
  a = ids.T.astype(jnp.int32)
  u = a & (_TBLK - 1)
  a2 = (a - u) + ((u & (_TQ - 1)) << 2) + (u >> 11)
  return a2.reshape(S, 4, B // 4).transpose(0, 2, 1).reshape(N)


def _mask_body(a_ref, b_ref, c_ref, d_ref, oa_ref, ob_ref, oc_ref, od_ref):
  oa_ref[...] = a_ref[...] == 0
  ob_ref[...] = b_ref[...] == 0
  oc_ref[...] = c_ref[...] == 0
  od_ref[...] = d_ref[...] == 0


def kernel(sources_input_ids, sources_attention_mask,
           hypotheses_input_ids, hypotheses_attention_mask,
           ref0_input_ids, ref0_attention_mask,
           ref1_input_ids, ref1_attention_mask,
           W_src, W_tgt):
  idx_src = _prep_idx(sources_input_ids)
  idx_hyp = _prep_idx(hypotheses_input_ids)
  idx_r0 = _prep_idx(ref0_input_ids)
  idx_r1 = _prep_idx(ref1_input_ids)

  ws = _transpose_table(W_src.T).reshape(_VPAD, D)
  wt = _transpose_table(W_tgt.T).reshape(_VPAD, D)

  gathered = _sc_gather()(ws, wt, idx_src, idx_hyp, idx_r0, idx_r1)
  outt = _transpose_out(gathered.reshape(4, S, B // 4, 4 * D))

  embedded_sources = outt[0].transpose(2, 0, 1)
  embedded_hypotheses = outt[1].transpose(2, 0, 1)
  embedded_ref0 = outt[2].transpose(2, 0, 1)
  embedded_ref1 = outt[3].transpose(2, 0, 1)

  inv = pl.pallas_call(
      _mask_body,
      out_shape=[jax.ShapeDtypeStruct((S, B), jnp.bool_)] * 4,
  )(sources_attention_mask.T, hypotheses_attention_mask.T,
    ref0_attention_mask.T, ref1_attention_mask.T)

  return (embedded_sources, embedded_hypotheses, embedded_ref0, embedded_ref1,
          inv[0].T, inv[1].T, inv[2].T, inv[3].T)


# speedup vs baseline: 2.1747x; 1.1972x over previous
import functools

import jax
import jax.numpy as jnp
from jax import lax
from jax.experimental import pallas as pl
from jax.experimental.pallas import tpu as pltpu
from jax.experimental.pallas import tpu_sc as plsc

B, S, V, D = 4096, 50, 1000000, 32
N = B * S

_info = plsc.get_sparse_core_info()
NC, NS = _info.num_cores, _info.num_subcores
NW = NC * NS
PER_W = N // NW
CH = 1600
NCH = PER_W // CH


def _sc_gather():
  mesh = plsc.VectorSubcoreMesh(core_axis_name="c", subcore_axis_name="s")

  @functools.partial(
      pl.kernel,
      mesh=mesh,
      compiler_params=pltpu.CompilerParams(use_tc_tiling_on_sc=False),
      out_type=jax.ShapeDtypeStruct((4 * N, D), jnp.float32),
      scratch_types=[
          pltpu.VMEM((CH,), jnp.int32),
          pltpu.VMEM((CH,), jnp.int32),
          pltpu.VMEM((CH, D), jnp.float32),
          pltpu.VMEM((CH, D), jnp.float32),
          pltpu.SemaphoreType.DMA,
          pltpu.SemaphoreType.DMA,
          pltpu.SemaphoreType.DMA,
      ],
  )
  def k(w_src, w_tgt, idx_src, idx_hyp, idx_r0, idx_r1, out,
        idx_v0, idx_v1, rows_v0, rows_v1, gsem, wsem0, wsem1):
    wid = lax.axis_index("s") * NC + lax.axis_index("c")
    idx_v = (idx_v0, idx_v1)
    rows_v = (rows_v0, rows_v1)
    wsem = (wsem0, wsem1)

    segs = ((w_src, idx_src), (w_tgt, idx_hyp), (w_tgt, idx_r0),
            (w_tgt, idx_r1))
    writes = [None, None]
    step = 0
    for seg, (table, idx_hbm) in enumerate(segs):
      for c in range(NCH):
        b = step % 2
        base = wid * PER_W + c * CH
        if writes[b] is not None:
          writes[b].wait()
        pltpu.sync_copy(idx_hbm.at[pl.ds(base, CH)], idx_v[b])
        pltpu.async_copy(table.at[idx_v[b]], rows_v[b], gsem).wait()
        writes[b] = pltpu.async_copy(
            rows_v[b], out.at[pl.ds(seg * N + base, CH)], wsem[b])
        step += 1
    for w in writes:
      if w is not None:
        w.wait()

  return k


_TBLK = 32768
_TQ = _TBLK // 4
_TQSH = (_TQ - 1).bit_length()
_TGRID = (V + _TBLK - 1) // _TBLK
_VPAD = _TGRID * _TBLK


def _table_t_body(wt_ref, out_ref):
  eye = jnp.eye(D, dtype=jnp.float32)
  parts = [
      lax.dot_general(wt_ref[:, c * _TQ:(c + 1) * _TQ], eye,
                      (((0,), (0,)), ((), ())),
                      preferred_element_type=jnp.float32)
      for c in range(4)
  ]
  out_ref[...] = jnp.concatenate(parts, axis=1)


def _transpose_table(wt):
  return pl.pallas_call(
      _table_t_body,
      grid=(_TGRID,),
      in_specs=[pl.BlockSpec((D, _TBLK), lambda j: (0, j))],
      out_specs=pl.BlockSpec((_TQ, 4 * D), lambda j: (j, 0)),
      out_shape=jax.ShapeDtypeStruct((_VPAD // 4, 4 * D), jnp.float32),
  )(wt)


def _out_t_body(g_ref, o0_ref, o1_ref, o2_ref, o3_ref):
  eye = jnp.eye(D, dtype=jnp.float32)
  for i, o_ref in enumerate((o0_ref, o1_ref, o2_ref, o3_ref)):
    g = g_ref[i, 0]
    parts = [
        lax.dot_general(eye, g[:, j * D:(j + 1) * D],
                        (((1,), (1,)), ((), ())),
                        preferred_element_type=jnp.float32)
        for j in range(4)
    ]
    o_ref[0] = jnp.concatenate(parts, axis=1)


def _transpose_out(g4):
  return pl.pallas_call(
      _out_t_body,
      grid=(S,),
      in_specs=[pl.BlockSpec((4, 1, B // 4, 4 * D), lambda s: (0, s, 0, 0))],
      out_specs=[pl.BlockSpec((1, D, B), lambda s: (s, 0, 0))] * 4,
      out_shape=[jax.ShapeDtypeStruct((S, D, B), jnp.float32)] * 4,
  )(g4)


def _prep_idx(ids):
  a = ids.T.astype(jnp.int32)
  u = a & (_TBLK - 1)
  a2 = (a - u) + ((u & (_TQ - 1)) << 2) + (u >> _TQSH)
  return a2.reshape(S, 4, B // 4).transpose(0, 2, 1).reshape(N)


def _mask_body(a_ref, b_ref, c_ref, d_ref, oa_ref, ob_ref, oc_ref, od_ref):
  oa_ref[...] = a_ref[...] == 0
  ob_ref[...] = b_ref[...] == 0
  oc_ref[...] = c_ref[...] == 0
  od_ref[...] = d_ref[...] == 0


def kernel(sources_input_ids, sources_attention_mask,
           hypotheses_input_ids, hypotheses_attention_mask,
           ref0_input_ids, ref0_attention_mask,
           ref1_input_ids, ref1_attention_mask,
           W_src, W_tgt):
  idx_src = _prep_idx(sources_input_ids)
  idx_hyp = _prep_idx(hypotheses_input_ids)
  idx_r0 = _prep_idx(ref0_input_ids)
  idx_r1 = _prep_idx(ref1_input_ids)

  ws = _transpose_table(W_src.T).reshape(_VPAD, D)
  wt = _transpose_table(W_tgt.T).reshape(_VPAD, D)

  gathered = _sc_gather()(ws, wt, idx_src, idx_hyp, idx_r0, idx_r1)
  o0, o1, o2, o3 = _transpose_out(gathered.reshape(4, S, B // 4, 4 * D))

  embedded_sources = o0.transpose(2, 0, 1)
  embedded_hypotheses = o1.transpose(2, 0, 1)
  embedded_ref0 = o2.transpose(2, 0, 1)
  embedded_ref1 = o3.transpose(2, 0, 1)

  inv = pl.pallas_call(
      _mask_body,
      out_shape=[jax.ShapeDtypeStruct((S, B), jnp.bool_)] * 4,
  )(sources_attention_mask.T, hypotheses_attention_mask.T,
    ref0_attention_mask.T, ref1_attention_mask.T)

  return (embedded_sources, embedded_hypotheses, embedded_ref0, embedded_ref1,
          inv[0].T, inv[1].T, inv[2].T, inv[3].T)


# speedup vs baseline: 2.6010x; 1.1960x over previous
import functools

import jax
import jax.numpy as jnp
from jax import lax
from jax.experimental import pallas as pl
from jax.experimental.pallas import tpu as pltpu
from jax.experimental.pallas import tpu_sc as plsc

B, S, V, D = 4096, 50, 1000000, 32
N = B * S

_info = plsc.get_sparse_core_info()
NC, NS = _info.num_cores, _info.num_subcores
NW = NC * NS
PER_W = N // NW
CH = 1600
NCH = PER_W // CH


def _sc_gather():
  mesh = plsc.VectorSubcoreMesh(core_axis_name="c", subcore_axis_name="s")

  @functools.partial(
      pl.kernel,
      mesh=mesh,
      compiler_params=pltpu.CompilerParams(use_tc_tiling_on_sc=False,
                                           needs_layout_passes=False),
      out_type=jax.ShapeDtypeStruct((4 * N, D), jnp.float32),
      scratch_types=[
          pltpu.VMEM((CH,), jnp.int32),
          pltpu.VMEM((CH,), jnp.int32),
          pltpu.VMEM((CH, D), jnp.float32),
          pltpu.VMEM((CH, D), jnp.float32),
          pltpu.SemaphoreType.DMA,
          pltpu.SemaphoreType.DMA,
          pltpu.SemaphoreType.DMA,
      ],
  )
  def k(w_src, w_tgt, idx_hbm, out,
        idx_v0, idx_v1, rows_v0, rows_v1, gsem, wsem0, wsem1):
    wid = lax.axis_index("s") * NC + lax.axis_index("c")
    idx_v = (idx_v0, idx_v1)
    rows_v = (rows_v0, rows_v1)
    wsem = (wsem0, wsem1)

    tables = (w_src, w_tgt, w_tgt, w_tgt)
    writes = [None, None]
    step = 0
    for seg, table in enumerate(tables):
      for c in range(NCH):
        b = step % 2
        base = seg * N + wid * PER_W + c * CH
        if writes[b] is not None:
          writes[b].wait()
        pltpu.sync_copy(idx_hbm.at[pl.ds(base, CH)], idx_v[b])
        pltpu.async_copy(table.at[idx_v[b]], rows_v[b], gsem).wait()
        writes[b] = pltpu.async_copy(
            rows_v[b], out.at[pl.ds(base, CH)], wsem[b])
        step += 1
    for w in writes:
      if w is not None:
        w.wait()

  return k


_PQ = 1024
_PCH = 4 * S * (B // _PQ) // NW


def _sc_prep_idx():
  mesh = plsc.VectorSubcoreMesh(core_axis_name="c", subcore_axis_name="s")

  @functools.partial(
      pl.kernel,
      mesh=mesh,
      compiler_params=pltpu.CompilerParams(use_tc_tiling_on_sc=False,
                                           needs_layout_passes=False),
      out_type=jax.ShapeDtypeStruct((4 * N,), jnp.int32),
      scratch_types=[
          pltpu.VMEM((_PQ,), jnp.int32),
          pltpu.VMEM((_PQ,), jnp.int32),
          pltpu.VMEM((_PQ,), jnp.int32),
      ],
  )
  def k(ids_hbm, out, raw_v, fix_v, out_v):
    wid = lax.axis_index("s") * NC + lax.axis_index("c")

    iota = lax.iota(jnp.int32, 16)
    perm16 = ((iota & 3) << 8) + (iota >> 2)
    qrun = _PQ // 4

    def chunk(t, _):
      gi = wid * _PCH + t
      seg = gi // (_PCH * NW // 4)
      rem = gi % (_PCH * NW // 4)
      s = rem // (B // _PQ)
      q = rem % (B // _PQ)
      off = seg * N + s * B + q * _PQ

      for j in range(4):
        pltpu.sync_copy(
            ids_hbm.at[pl.ds(seg * N + s * B + j * (B // 4) + q * qrun, qrun)],
            raw_v.at[pl.ds(j * qrun, qrun)])

      for g in range(_PQ // 16):
        v = raw_v[pl.ds(g * 16, 16)]
        u = v & (_TBLK - 1)
        fix_v[pl.ds(g * 16, 16)] = (
            (v - u) + ((u & (_TQ - 1)) << 2) + (u >> _TQSH))

      for g in range(_PQ // 16):
        out_v[pl.ds(g * 16, 16)] = plsc.load_gather(fix_v, [perm16 + 4 * g])

      pltpu.sync_copy(out_v, out.at[pl.ds(off, _PQ)])
      return 0

    lax.fori_loop(0, _PCH, chunk, 0)

  return k


_TBLK = 32768
_TQ = _TBLK // 4
_TQSH = (_TQ - 1).bit_length()
_TGRID = (V + _TBLK - 1) // _TBLK
_VPAD = _TGRID * _TBLK


def _table_t_body(wt_ref, out_ref):
  eye = jnp.eye(D, dtype=jnp.float32)
  parts = [
      lax.dot_general(wt_ref[:, c * _TQ:(c + 1) * _TQ], eye,
                      (((0,), (0,)), ((), ())),
                      preferred_element_type=jnp.float32)
      for c in range(4)
  ]
  out_ref[...] = jnp.concatenate(parts, axis=1)


def _transpose_table(wt):
  return pl.pallas_call(
      _table_t_body,
      grid=(_TGRID,),
      in_specs=[pl.BlockSpec((D, _TBLK), lambda j: (0, j))],
      out_specs=pl.BlockSpec((_TQ, 4 * D), lambda j: (j, 0)),
      out_shape=jax.ShapeDtypeStruct((_VPAD // 4, 4 * D), jnp.float32),
  )(wt)


def _out_t_body(g_ref, o0_ref, o1_ref, o2_ref, o3_ref):
  eye = jnp.eye(D, dtype=jnp.float32)
  for i, o_ref in enumerate((o0_ref, o1_ref, o2_ref, o3_ref)):
    g = g_ref[i, 0]
    parts = [
        lax.dot_general(eye, g[:, j * D:(j + 1) * D],
                        (((1,), (1,)), ((), ())),
                        preferred_element_type=jnp.float32)
        for j in range(4)
    ]
    o_ref[0] = jnp.concatenate(parts, axis=1)


def _transpose_out(g4):
  return pl.pallas_call(
      _out_t_body,
      grid=(S,),
      in_specs=[pl.BlockSpec((4, 1, B // 4, 4 * D), lambda s: (0, s, 0, 0))],
      out_specs=[pl.BlockSpec((1, D, B), lambda s: (s, 0, 0))] * 4,
      out_shape=[jax.ShapeDtypeStruct((S, D, B), jnp.float32)] * 4,
  )(g4)


def _mask_body(a_ref, b_ref, c_ref, d_ref, oa_ref, ob_ref, oc_ref, od_ref):
  oa_ref[...] = a_ref[...] == 0
  ob_ref[...] = b_ref[...] == 0
  oc_ref[...] = c_ref[...] == 0
  od_ref[...] = d_ref[...] == 0


def kernel(sources_input_ids, sources_attention_mask,
           hypotheses_input_ids, hypotheses_attention_mask,
           ref0_input_ids, ref0_attention_mask,
           ref1_input_ids, ref1_attention_mask,
           W_src, W_tgt):
  ids_flat = jnp.concatenate([
      sources_input_ids.T.reshape(N).astype(jnp.int32),
      hypotheses_input_ids.T.reshape(N).astype(jnp.int32),
      ref0_input_ids.T.reshape(N).astype(jnp.int32),
      ref1_input_ids.T.reshape(N).astype(jnp.int32),
  ])
  idx = _sc_prep_idx()(ids_flat)

  ws = _transpose_table(W_src.T).reshape(_VPAD, D)
  wt = _transpose_table(W_tgt.T).reshape(_VPAD, D)

  gathered = _sc_gather()(ws, wt, idx)
  o0, o1, o2, o3 = _transpose_out(gathered.reshape(4, S, B // 4, 4 * D))

  embedded_sources = o0.transpose(2, 0, 1)
  embedded_hypotheses = o1.transpose(2, 0, 1)
  embedded_ref0 = o2.transpose(2, 0, 1)
  embedded_ref1 = o3.transpose(2, 0, 1)

  inv = pl.pallas_call(
      _mask_body,
      out_shape=[jax.ShapeDtypeStruct((S, B), jnp.bool_)] * 4,
  )(sources_attention_mask.T, hypotheses_attention_mask.T,
    ref0_attention_mask.T, ref1_attention_mask.T)

  return (embedded_sources, embedded_hypotheses, embedded_ref0, embedded_ref1,
          inv[0].T, inv[1].T, inv[2].T, inv[3].T)


# speedup vs baseline: 2.6735x; 1.0279x over previous
import functools

import jax
import jax.numpy as jnp
from jax import lax
from jax.experimental import pallas as pl
from jax.experimental.pallas import tpu as pltpu
from jax.experimental.pallas import tpu_sc as plsc

B, S, V, D = 4096, 50, 1000000, 32
N = B * S

_info = plsc.get_sparse_core_info()
NC, NS = _info.num_cores, _info.num_subcores
NW = NC * NS
PER_W = N // NW
CH = 1600
NCH = PER_W // CH


def _sc_gather(ntok, idx_off):
  per_w = ntok // NW
  nch = per_w // CH
  mesh = plsc.VectorSubcoreMesh(core_axis_name="c", subcore_axis_name="s")

  @functools.partial(
      pl.kernel,
      mesh=mesh,
      compiler_params=pltpu.CompilerParams(use_tc_tiling_on_sc=False,
                                           needs_layout_passes=False),
      out_type=jax.ShapeDtypeStruct((ntok, D), jnp.float32),
      scratch_types=[
          pltpu.VMEM((CH,), jnp.int32),
          pltpu.VMEM((CH,), jnp.int32),
          pltpu.VMEM((CH, D), jnp.float32),
          pltpu.VMEM((CH, D), jnp.float32),
          pltpu.SemaphoreType.DMA,
          pltpu.SemaphoreType.DMA,
          pltpu.SemaphoreType.DMA,
      ],
  )
  def k(table, idx_hbm, out,
        idx_v0, idx_v1, rows_v0, rows_v1, gsem, wsem0, wsem1):
    wid = lax.axis_index("s") * NC + lax.axis_index("c")
    idx_v = (idx_v0, idx_v1)
    rows_v = (rows_v0, rows_v1)
    wsem = (wsem0, wsem1)

    writes = [None, None]
    for c in range(nch):
      b = c % 2
      base = wid * per_w + c * CH
      if writes[b] is not None:
        writes[b].wait()
      pltpu.sync_copy(idx_hbm.at[pl.ds(idx_off + base, CH)], idx_v[b])
      pltpu.async_copy(table.at[idx_v[b]], rows_v[b], gsem).wait()
      writes[b] = pltpu.async_copy(
          rows_v[b], out.at[pl.ds(base, CH)], wsem[b])
    for w in writes:
      if w is not None:
        w.wait()

  return k


_PQ = 1024
_PCH = 4 * S * (B // _PQ) // NW


def _sc_prep_idx():
  mesh = plsc.VectorSubcoreMesh(core_axis_name="c", subcore_axis_name="s")

  @functools.partial(
      pl.kernel,
      mesh=mesh,
      compiler_params=pltpu.CompilerParams(use_tc_tiling_on_sc=False,
                                           needs_layout_passes=False),
      out_type=jax.ShapeDtypeStruct((4 * N,), jnp.int32),
      scratch_types=[
          pltpu.VMEM((_PQ,), jnp.int32),
          pltpu.VMEM((_PQ,), jnp.int32),
          pltpu.VMEM((_PQ,), jnp.int32),
      ],
  )
  def k(ids_hbm, out, raw_v, fix_v, out_v):
    wid = lax.axis_index("s") * NC + lax.axis_index("c")

    iota = lax.iota(jnp.int32, 16)
    perm16 = ((iota & 3) << 8) + (iota >> 2)
    qrun = _PQ // 4

    def chunk(t, _):
      gi = wid * _PCH + t
      seg = gi // (_PCH * NW // 4)
      rem = gi % (_PCH * NW // 4)
      s = rem // (B // _PQ)
      q = rem % (B // _PQ)
      off = seg * N + s * B + q * _PQ

      for j in range(4):
        pltpu.sync_copy(
            ids_hbm.at[pl.ds(seg * N + s * B + j * (B // 4) + q * qrun, qrun)],
            raw_v.at[pl.ds(j * qrun, qrun)])

      for g in range(_PQ // 16):
        v = raw_v[pl.ds(g * 16, 16)]
        u = v & (_TBLK - 1)
        fix_v[pl.ds(g * 16, 16)] = (
            (v - u) + ((u & (_TQ - 1)) << 2) + (u >> _TQSH))

      for g in range(_PQ // 16):
        out_v[pl.ds(g * 16, 16)] = plsc.load_gather(fix_v, [perm16 + 4 * g])

      pltpu.sync_copy(out_v, out.at[pl.ds(off, _PQ)])
      return 0

    lax.fori_loop(0, _PCH, chunk, 0)

  return k


_TBLK = 32768
_TQ = _TBLK // 4
_TQSH = (_TQ - 1).bit_length()
_TGRID = (V + _TBLK - 1) // _TBLK
_VPAD = _TGRID * _TBLK


def _table_t_body(wt_ref, out_ref):
  eye = jnp.eye(D, dtype=jnp.float32)
  parts = [
      lax.dot_general(wt_ref[:, c * _TQ:(c + 1) * _TQ], eye,
                      (((0,), (0,)), ((), ())),
                      preferred_element_type=jnp.float32)
      for c in range(4)
  ]
  out_ref[...] = jnp.concatenate(parts, axis=1)


def _transpose_table(wt):
  return pl.pallas_call(
      _table_t_body,
      grid=(_TGRID,),
      in_specs=[pl.BlockSpec((D, _TBLK), lambda j: (0, j))],
      out_specs=pl.BlockSpec((_TQ, 4 * D), lambda j: (j, 0)),
      out_shape=jax.ShapeDtypeStruct((_VPAD // 4, 4 * D), jnp.float32),
  )(wt)


def _out_t_body(gs_ref, gt_ref, o0_ref, o1_ref, o2_ref, o3_ref):
  eye = jnp.eye(D, dtype=jnp.float32)
  blocks = [gs_ref[0]] + [gt_ref[i, 0] for i in range(3)]
  for g, o_ref in zip(blocks, (o0_ref, o1_ref, o2_ref, o3_ref)):
    parts = [
        lax.dot_general(eye, g[:, j * D:(j + 1) * D],
                        (((1,), (1,)), ((), ())),
                        preferred_element_type=jnp.float32)
        for j in range(4)
    ]
    o_ref[0] = jnp.concatenate(parts, axis=1)


def _transpose_out(gs, gt):
  return pl.pallas_call(
      _out_t_body,
      grid=(S,),
      in_specs=[
          pl.BlockSpec((1, B // 4, 4 * D), lambda s: (s, 0, 0)),
          pl.BlockSpec((3, 1, B // 4, 4 * D), lambda s: (0, s, 0, 0)),
      ],
      out_specs=[pl.BlockSpec((1, D, B), lambda s: (s, 0, 0))] * 4,
      out_shape=[jax.ShapeDtypeStruct((S, D, B), jnp.float32)] * 4,
  )(gs, gt)


def _mask_body(a_ref, b_ref, c_ref, d_ref, oa_ref, ob_ref, oc_ref, od_ref):
  oa_ref[...] = a_ref[...] == 0
  ob_ref[...] = b_ref[...] == 0
  oc_ref[...] = c_ref[...] == 0
  od_ref[...] = d_ref[...] == 0


def kernel(sources_input_ids, sources_attention_mask,
           hypotheses_input_ids, hypotheses_attention_mask,
           ref0_input_ids, ref0_attention_mask,
           ref1_input_ids, ref1_attention_mask,
           W_src, W_tgt):
  ids_flat = jnp.concatenate([
      sources_input_ids.T.reshape(N).astype(jnp.int32),
      hypotheses_input_ids.T.reshape(N).astype(jnp.int32),
      ref0_input_ids.T.reshape(N).astype(jnp.int32),
      ref1_input_ids.T.reshape(N).astype(jnp.int32),
  ])
  idx = _sc_prep_idx()(ids_flat)

  wt = _transpose_table(W_tgt.T).reshape(_VPAD, D)
  gtgt = _sc_gather(3 * N, N)(wt, idx)
  ws = _transpose_table(W_src.T).reshape(_VPAD, D)
  gsrc = _sc_gather(N, 0)(ws, idx)

  o0, o1, o2, o3 = _transpose_out(
      gsrc.reshape(S, B // 4, 4 * D), gtgt.reshape(3, S, B // 4, 4 * D))

  embedded_sources = o0.transpose(2, 0, 1)
  embedded_hypotheses = o1.transpose(2, 0, 1)
  embedded_ref0 = o2.transpose(2, 0, 1)
  embedded_ref1 = o3.transpose(2, 0, 1)

  inv = pl.pallas_call(
      _mask_body,
      out_shape=[jax.ShapeDtypeStruct((S, B), jnp.bool_)] * 4,
  )(sources_attention_mask.T, hypotheses_attention_mask.T,
    ref0_attention_mask.T, ref1_attention_mask.T)

  return (embedded_sources, embedded_hypotheses, embedded_ref0, embedded_ref1,
          inv[0].T, inv[1].T, inv[2].T, inv[3].T)


# speedup vs baseline: 2.6787x; 1.0020x over previous
import functools

import jax
import jax.numpy as jnp
from jax import lax
from jax.experimental import pallas as pl
from jax.experimental.pallas import tpu as pltpu
from jax.experimental.pallas import tpu_sc as plsc

B, S, V, D = 4096, 50, 1000000, 32
N = B * S

_info = plsc.get_sparse_core_info()
NC, NS = _info.num_cores, _info.num_subcores
NW = NC * NS
CH = 1600


def _sc_gather(ntok, idx_off):
  per_w = ntok // NW
  nch = per_w // CH
  mesh = plsc.VectorSubcoreMesh(core_axis_name="c", subcore_axis_name="s")

  @functools.partial(
      pl.kernel,
      mesh=mesh,
      compiler_params=pltpu.CompilerParams(use_tc_tiling_on_sc=False,
                                           needs_layout_passes=False),
      out_type=jax.ShapeDtypeStruct((ntok, D), jnp.float32),
      scratch_types=[
          pltpu.VMEM((CH,), jnp.int32),
          pltpu.VMEM((CH,), jnp.int32),
          pltpu.VMEM((CH, D), jnp.float32),
          pltpu.VMEM((CH, D), jnp.float32),
          pltpu.SemaphoreType.DMA,
          pltpu.SemaphoreType.DMA,
          pltpu.SemaphoreType.DMA,
      ],
  )
  def k(table, idx_hbm, out,
        idx_v0, idx_v1, rows_v0, rows_v1, gsem, wsem0, wsem1):
    wid = lax.axis_index("s") * NC + lax.axis_index("c")
    idx_v = (idx_v0, idx_v1)
    rows_v = (rows_v0, rows_v1)
    wsem = (wsem0, wsem1)

    writes = [None, None]
    for c in range(nch):
      b = c % 2
      base = wid * per_w + c * CH
      if writes[b] is not None:
        writes[b].wait()
      pltpu.sync_copy(idx_hbm.at[pl.ds(idx_off + base, CH)], idx_v[b])
      pltpu.async_copy(table.at[idx_v[b]], rows_v[b], gsem).wait()
      writes[b] = pltpu.async_copy(
          rows_v[b], out.at[pl.ds(base, CH)], wsem[b])
    for w in writes:
      if w is not None:
        w.wait()

  return k


_PQ = 1024
_PCH = 4 * S * (B // _PQ) // NW


def _sc_prep_idx():
  mesh = plsc.VectorSubcoreMesh(core_axis_name="c", subcore_axis_name="s")

  @functools.partial(
      pl.kernel,
      mesh=mesh,
      compiler_params=pltpu.CompilerParams(use_tc_tiling_on_sc=False,
                                           needs_layout_passes=False),
      out_type=jax.ShapeDtypeStruct((4 * N,), jnp.int32),
      scratch_types=[
          pltpu.VMEM((_PQ,), jnp.int32),
          pltpu.VMEM((_PQ,), jnp.int32),
          pltpu.VMEM((_PQ,), jnp.int32),
      ],
  )
  def k(ids_hbm, out, raw_v, fix_v, out_v):
    wid = lax.axis_index("s") * NC + lax.axis_index("c")

    iota = lax.iota(jnp.int32, 16)
    perm16 = ((iota & 3) << 8) + (iota >> 2)
    qrun = _PQ // 4

    def chunk(t, _):
      gi = wid * _PCH + t
      seg = gi // (_PCH * NW // 4)
      rem = gi % (_PCH * NW // 4)
      s = rem // (B // _PQ)
      q = rem % (B // _PQ)
      off = seg * N + s * B + q * _PQ

      for j in range(4):
        pltpu.sync_copy(
            ids_hbm.at[pl.ds(seg * N + s * B + j * (B // 4) + q * qrun, qrun)],
            raw_v.at[pl.ds(j * qrun, qrun)])

      for g in range(_PQ // 16):
        v = raw_v[pl.ds(g * 16, 16)]
        u = v & (_TBLK - 1)
        fix_v[pl.ds(g * 16, 16)] = (
            (v - u) + ((u & (_TQ - 1)) << 2) + (u >> _TQSH))

      for g in range(_PQ // 16):
        out_v[pl.ds(g * 16, 16)] = plsc.load_gather(fix_v, [perm16 + 4 * g])

      pltpu.sync_copy(out_v, out.at[pl.ds(off, _PQ)])
      return 0

    lax.fori_loop(0, _PCH, chunk, 0)

  return k


_TBLK = 32768
_TQ = _TBLK // 4
_TQSH = (_TQ - 1).bit_length()
_TGRID = (V + _TBLK - 1) // _TBLK
_VPAD = _TGRID * _TBLK


def _table_t_body(wt_ref, out_ref):
  eye = jnp.eye(D, dtype=jnp.float32)
  parts = [
      lax.dot_general(wt_ref[:, c * _TQ:(c + 1) * _TQ], eye,
                      (((0,), (0,)), ((), ())),
                      preferred_element_type=jnp.float32)
      for c in range(4)
  ]
  out_ref[...] = jnp.concatenate(parts, axis=1)


def _transpose_table(wt):
  return pl.pallas_call(
      _table_t_body,
      grid=(_TGRID,),
      in_specs=[pl.BlockSpec((D, _TBLK), lambda j: (0, j))],
      out_specs=pl.BlockSpec((_TQ, 4 * D), lambda j: (j, 0)),
      out_shape=jax.ShapeDtypeStruct((_VPAD // 4, 4 * D), jnp.float32),
  )(wt)


def _out_t_body(gs_ref, gt_ref, o0_ref, o1_ref, o2_ref, o3_ref):
  eye = jnp.eye(D, dtype=jnp.float32)
  blocks = [gs_ref[0]] + [gt_ref[i, 0] for i in range(3)]
  for g, o_ref in zip(blocks, (o0_ref, o1_ref, o2_ref, o3_ref)):
    parts = [
        lax.dot_general(eye, g[:, j * D:(j + 1) * D],
                        (((1,), (1,)), ((), ())),
                        preferred_element_type=jnp.float32)
        for j in range(4)
    ]
    o_ref[0] = jnp.concatenate(parts, axis=1)


def _transpose_out(gs, gt):
  return pl.pallas_call(
      _out_t_body,
      grid=(S,),
      in_specs=[
          pl.BlockSpec((1, B // 4, 4 * D), lambda s: (s, 0, 0)),
          pl.BlockSpec((3, 1, B // 4, 4 * D), lambda s: (0, s, 0, 0)),
      ],
      out_specs=[pl.BlockSpec((1, D, B), lambda s: (s, 0, 0))] * 4,
      out_shape=[jax.ShapeDtypeStruct((S, D, B), jnp.float32)] * 4,
  )(gs, gt)


def _mask_body(a_ref, b_ref, c_ref, d_ref, oa_ref, ob_ref, oc_ref, od_ref):
  oa_ref[...] = a_ref[...] == 0
  ob_ref[...] = b_ref[...] == 0
  oc_ref[...] = c_ref[...] == 0
  od_ref[...] = d_ref[...] == 0


def kernel(sources_input_ids, sources_attention_mask,
           hypotheses_input_ids, hypotheses_attention_mask,
           ref0_input_ids, ref0_attention_mask,
           ref1_input_ids, ref1_attention_mask,
           W_src, W_tgt):
  ids_flat = jnp.concatenate([
      sources_input_ids.T.reshape(N).astype(jnp.int32),
      hypotheses_input_ids.T.reshape(N).astype(jnp.int32),
      ref0_input_ids.T.reshape(N).astype(jnp.int32),
      ref1_input_ids.T.reshape(N).astype(jnp.int32),
  ])
  idx = _sc_prep_idx()(ids_flat)

  wt = _transpose_table(W_tgt.T).reshape(_VPAD, D)
  gtgt = _sc_gather(3 * N, N)(wt, idx)
  ws = _transpose_table(W_src.T).reshape(_VPAD, D)
  gsrc = _sc_gather(N, 0)(ws, idx)

  o0, o1, o2, o3 = _transpose_out(
      gsrc.reshape(S, B // 4, 4 * D), gtgt.reshape(3, S, B // 4, 4 * D))

  embedded_sources = o0.transpose(2, 0, 1)
  embedded_hypotheses = o1.transpose(2, 0, 1)
  embedded_ref0 = o2.transpose(2, 0, 1)
  embedded_ref1 = o3.transpose(2, 0, 1)

  inv = pl.pallas_call(
      _mask_body,
      out_shape=[jax.ShapeDtypeStruct((S, B), jnp.bool_)] * 4,
  )(sources_attention_mask.T, hypotheses_attention_mask.T,
    ref0_attention_mask.T, ref1_attention_mask.T)

  return (embedded_sources, embedded_hypotheses, embedded_ref0, embedded_ref1,
          inv[0].T, inv[1].T, inv[2].T, inv[3].T)
